# Initial kernel scaffold; baseline (speedup 1.0000x reference)
import jax
import jax.numpy as jnp
from jax.experimental import pallas as pl


def kernel(x, wia, whh, b):
    raise NotImplementedError("write your pallas kernel here")


# speedup vs baseline: 1.4660x; 1.4660x over previous
import functools

import jax
import jax.numpy as jnp
from jax.experimental import pallas as pl
from jax.experimental.pallas import tpu as pltpu


def _sigmoid(x):
    return 0.5 * jnp.tanh(0.5 * x) + 0.5


def _bilstm_mean_relu_kernel(x_ref, wia_ref, whh_ref, b_ref, out_ref, gx_ref,
                             *, H, unroll):
    B_blk, T, _ = x_ref.shape
    _, _, G = gx_ref.shape
    GP = G // 4
    inv_T = 1.0 / T

    wia = wia_ref[...]
    bias = b_ref[...]

    lane = jax.lax.broadcasted_iota(jnp.int32, (1, G), 1) % GP
    bwd_mask = jnp.logical_and(lane >= H, lane < 2 * H)

    for k in range(T // 2):
        t2 = T - 1 - k
        p1 = jnp.dot(x_ref[:, k, :], wia,
                     preferred_element_type=jnp.float32) + bias
        p2 = jnp.dot(x_ref[:, t2, :], wia,
                     preferred_element_type=jnp.float32) + bias
        gx_ref[k] = jnp.where(bwd_mask, p2, p1)
        gx_ref[t2] = jnp.where(bwd_mask, p1, p2)

    whh = whh_ref[...]

    def step(t, carry):
        h, c, acc = carry
        gates = gx_ref[t] + jnp.dot(h, whh, preferred_element_type=jnp.float32)
        i = _sigmoid(gates[:, 0 * GP:1 * GP])
        f = _sigmoid(gates[:, 1 * GP:2 * GP])
        g = jnp.tanh(gates[:, 2 * GP:3 * GP])
        o = _sigmoid(gates[:, 3 * GP:4 * GP])
        c = f * c + i * g
        h = o * jnp.tanh(c)
        return h, c, acc + h

    h = jnp.zeros((B_blk, GP), jnp.float32)
    c = jnp.zeros((B_blk, GP), jnp.float32)
    acc = jnp.zeros((B_blk, GP), jnp.float32)
    h, c, acc = jax.lax.fori_loop(0, T, step, (h, c, acc), unroll=unroll)

    out_ref[...] = jnp.maximum(acc * inv_T, 0.0)


def kernel(x, wia, whh, b):
    B, T, D = x.shape
    GP = whh.shape[0]
    G = wia.shape[1]
    H = GP // 2

    Bp = max(8, -(-B // 8) * 8)
    B_BLK = min(Bp, 256)
    Bp = -(-Bp // B_BLK) * B_BLK
    if Bp != B:
        x = jnp.pad(x, ((0, Bp - B), (0, 0), (0, 0)))

    body = functools.partial(_bilstm_mean_relu_kernel, H=H, unroll=4)

    out = pl.pallas_call(
        body,
        out_shape=jax.ShapeDtypeStruct((Bp, GP), jnp.float32),
        grid=(Bp // B_BLK,),
        in_specs=[
            pl.BlockSpec((B_BLK, T, D), lambda i: (i, 0, 0)),
            pl.BlockSpec(wia.shape, lambda i: (0, 0)),
            pl.BlockSpec(whh.shape, lambda i: (0, 0)),
            pl.BlockSpec(b.shape, lambda i: (0, 0)),
        ],
        out_specs=pl.BlockSpec((B_BLK, GP), lambda i: (i, 0)),
        scratch_shapes=[pltpu.VMEM((T, B_BLK, G), jnp.float32)],
        compiler_params=pltpu.CompilerParams(
            dimension_semantics=("parallel",),
            vmem_limit_bytes=56 * 1024 * 1024,
        ),
    )(x, wia, whh, b)

    return out[:B, :GP]
